# initial kernel scaffold (unmeasured)
import jax
import jax.numpy as jnp
from jax import lax
from jax.experimental import pallas as pl
from jax.experimental.pallas import tpu as pltpu


def kernel(
    x,
):
    def body(*refs):
        pass

    out_shape = jax.ShapeDtypeStruct(..., jnp.float32)
    return pl.pallas_call(body, out_shape=out_shape)(...)


# baseline (device time: 15722 ns/iter reference)
import jax
import jax.numpy as jnp
from jax import lax
from jax.experimental import pallas as pl
from jax.experimental.pallas import tpu as pltpu

N_DEV = 16
BLOCK_M = 512


def kernel(x):
    m, n = x.shape
    assert m % BLOCK_M == 0
    k_steps = m // BLOCK_M

    def body(x_ref, out_ref, partial_ref, comm_ref, send_sems, recv_sems):
        k = pl.program_id(0)

        psum = jnp.sum(x_ref[:, :].astype(jnp.float32), axis=0, keepdims=True)

        @pl.when(k == 0)
        def _():
            partial_ref[:, :] = psum

        @pl.when(k > 0)
        def _():
            partial_ref[:, :] = partial_ref[:, :] + psum

        @pl.when(k == k_steps - 1)
        def _():
            my_pos = lax.axis_index("i")

            barrier_sem = pltpu.get_barrier_semaphore()
            for d in range(1, N_DEV):
                pl.semaphore_signal(
                    barrier_sem,
                    inc=1,
                    device_id=((my_pos + d) % N_DEV,),
                    device_id_type=pl.DeviceIdType.MESH,
                )
            pl.semaphore_wait(barrier_sem, N_DEV - 1)

            rdmas = []
            for d in range(1, N_DEV):
                rdma = pltpu.make_async_remote_copy(
                    src_ref=partial_ref,
                    dst_ref=comm_ref.at[d - 1],
                    send_sem=send_sems.at[d - 1],
                    recv_sem=recv_sems.at[d - 1],
                    device_id=((my_pos + d) % N_DEV,),
                    device_id_type=pl.DeviceIdType.MESH,
                )
                rdma.start()
                rdmas.append(rdma)

            acc = partial_ref[:, :]
            for d in range(1, N_DEV):
                rdmas[d - 1].wait_recv()
                acc = acc + comm_ref[d - 1, :, :]
            out_ref[:, :] = acc

            for d in range(1, N_DEV):
                rdmas[d - 1].wait_send()

    return pl.pallas_call(
        body,
        grid=(k_steps,),
        out_shape=jax.ShapeDtypeStruct((1, n), jnp.float32),
        in_specs=[
            pl.BlockSpec((BLOCK_M, n), lambda k: (k, 0), memory_space=pltpu.VMEM)
        ],
        out_specs=pl.BlockSpec((1, n), lambda k: (0, 0), memory_space=pltpu.VMEM),
        scratch_shapes=[
            pltpu.VMEM((1, n), jnp.float32),
            pltpu.VMEM((N_DEV - 1, 1, n), jnp.float32),
            pltpu.SemaphoreType.DMA((N_DEV - 1,)),
            pltpu.SemaphoreType.DMA((N_DEV - 1,)),
        ],
        compiler_params=pltpu.CompilerParams(
            collective_id=0,
            dimension_semantics=("arbitrary",),
        ),
    )(x)


# device time: 15214 ns/iter; 1.0334x vs baseline; 1.0334x over previous
import jax
import jax.numpy as jnp
from jax import lax
from jax.experimental import pallas as pl
from jax.experimental.pallas import tpu as pltpu

N_DEV = 16
BLOCK_M = 512


def kernel(x):
    m, n = x.shape
    assert m % BLOCK_M == 0
    k_steps = m // BLOCK_M

    def body(x_ref, out_ref, partial_ref, comm_ref, send_sems, recv_sems):
        k = pl.program_id(0)

        psum = jnp.sum(x_ref[:, :].astype(jnp.float32), axis=0, keepdims=True)

        @pl.when(k == 0)
        def _():
            partial_ref[:, :] = psum

        @pl.when(k > 0)
        def _():
            partial_ref[:, :] = partial_ref[:, :] + psum

        @pl.when(k == 0)
        def _():
            my_pos = lax.axis_index("i")
            barrier_sem = pltpu.get_barrier_semaphore()
            for d in range(1, N_DEV):
                pl.semaphore_signal(
                    barrier_sem,
                    inc=1,
                    device_id=((my_pos + d) % N_DEV,),
                    device_id_type=pl.DeviceIdType.MESH,
                )

        @pl.when(k == k_steps - 1)
        def _():
            my_pos = lax.axis_index("i")

            barrier_sem = pltpu.get_barrier_semaphore()
            pl.semaphore_wait(barrier_sem, N_DEV - 1)

            rdmas = []
            for d in range(1, N_DEV):
                rdma = pltpu.make_async_remote_copy(
                    src_ref=partial_ref,
                    dst_ref=comm_ref.at[d - 1],
                    send_sem=send_sems.at[d - 1],
                    recv_sem=recv_sems.at[d - 1],
                    device_id=((my_pos + d) % N_DEV,),
                    device_id_type=pl.DeviceIdType.MESH,
                )
                rdma.start()
                rdmas.append(rdma)

            acc = partial_ref[:, :]
            for d in range(1, N_DEV):
                rdmas[d - 1].wait_recv()
                acc = acc + comm_ref[d - 1, :, :]
            out_ref[:, :] = acc

            for d in range(1, N_DEV):
                rdmas[d - 1].wait_send()

    return pl.pallas_call(
        body,
        grid=(k_steps,),
        out_shape=jax.ShapeDtypeStruct((1, n), jnp.float32),
        in_specs=[
            pl.BlockSpec((BLOCK_M, n), lambda k: (k, 0), memory_space=pltpu.VMEM)
        ],
        out_specs=pl.BlockSpec((1, n), lambda k: (0, 0), memory_space=pltpu.VMEM),
        scratch_shapes=[
            pltpu.VMEM((1, n), jnp.float32),
            pltpu.VMEM((N_DEV - 1, 1, n), jnp.float32),
            pltpu.SemaphoreType.DMA((N_DEV - 1,)),
            pltpu.SemaphoreType.DMA((N_DEV - 1,)),
        ],
        compiler_params=pltpu.CompilerParams(
            collective_id=0,
            dimension_semantics=("arbitrary",),
        ),
    )(x)
